# Initial kernel scaffold; baseline (speedup 1.0000x reference)
import jax
import jax.numpy as jnp
from jax.experimental import pallas as pl


def kernel(x, edge_index, edge_attr, x_emb1, x_emb2, e_emb1, e_emb2, W1, b1, W2, b2, gamma, beta):
    raise NotImplementedError("write your pallas kernel here")


# speedup vs baseline: 13.3105x; 13.3105x over previous
import functools

import jax
import jax.numpy as jnp
from jax import lax
from jax.experimental import pallas as pl
from jax.experimental.pallas import tpu as pltpu
from jax.experimental.pallas import tpu_sc as plsc

_N = 10000
_E = 320000
_D = 128
_NC = 2
_NS = 16
_NW = _NC * _NS
_CH = 128
_NCH = 80
_NPH = 2
_CPP = _NCH // _NPH
_EPT = _NCH * _CH
_EPAD = _NW * _EPT
_RPT = 626
_NSLAB = _NS * _RPT
_NF = _NS * 10240
_FPT = _NF // _NS

def _sc_aggr_body(h_hbm, src_hbm, dst_hbm, z_hbm, out_hbm,
                  src_v, dst_v, buf_a, buf_b, slab, sem_a, sem_b):
    c = lax.axis_index("c")
    s = lax.axis_index("s")
    wid = c * _NS + s
    pltpu.sync_copy(z_hbm, slab.at[pl.ds(s * _RPT, _RPT)])
    plsc.subcore_barrier()

    for p in range(_NPH):
        pltpu.sync_copy(src_hbm.at[wid, p], src_v)
        pltpu.sync_copy(dst_hbm.at[wid, p], dst_v)
        pltpu.async_copy(h_hbm.at[src_v.at[0]], buf_a, sem_a)
        pltpu.async_copy(h_hbm.at[src_v.at[1]], buf_b, sem_b)

        def body(i, _):
            ja = 2 * i
            jb = 2 * i + 1
            pltpu.make_async_copy(h_hbm.at[src_v.at[ja]], buf_a, sem_a).wait()
            pltpu.sync_copy(buf_a, slab.at[dst_v.at[ja]], add=True)

            @pl.when(ja + 2 < _CPP)
            def _():
                pltpu.async_copy(h_hbm.at[src_v.at[ja + 2]], buf_a, sem_a)

            pltpu.make_async_copy(h_hbm.at[src_v.at[jb]], buf_b, sem_b).wait()
            pltpu.sync_copy(buf_b, slab.at[dst_v.at[jb]], add=True)

            @pl.when(jb + 2 < _CPP)
            def _():
                pltpu.async_copy(h_hbm.at[src_v.at[jb + 2]], buf_b, sem_b)

            return _

        lax.fori_loop(0, _CPP // 2, body, None)
    plsc.subcore_barrier()
    pltpu.sync_copy(slab.at[pl.ds(s * _RPT, _RPT)], out_hbm.at[wid])


def _sc_counts_body(dst_hbm, a0_hbm, a1_hbm, z_hbm, out_hbm,
                    dst_v, a0_v, a1_v, idx1_v, idx2_v, ones_v, slab):
    c = lax.axis_index("c")
    s = lax.axis_index("s")
    wid = c * _NS + s
    pltpu.sync_copy(dst_hbm.at[wid], dst_v)
    pltpu.sync_copy(a0_hbm.at[wid], a0_v)
    pltpu.sync_copy(a1_hbm.at[wid], a1_v)
    pltpu.sync_copy(z_hbm, slab.at[pl.ds(s * _FPT, _FPT)])
    for v in range(_CH // 16):
        ones_v[pl.ds(v * 16, 16)] = jnp.ones((16,), jnp.float32)
    plsc.subcore_barrier()

    def body(j, _):
        for v in range(_CH // 16):
            sl = pl.ds(v * 16, 16)
            d16 = dst_v[j, sl] * 16
            idx1_v[sl] = d16 + a0_v[j, sl]
            idx2_v[sl] = d16 + (a1_v[j, sl] + 8)
        pltpu.sync_copy(ones_v, slab.at[idx1_v], add=True)
        pltpu.sync_copy(ones_v, slab.at[idx2_v], add=True)
        return _

    lax.fori_loop(0, _NCH, body, None)
    plsc.subcore_barrier()
    pltpu.sync_copy(slab.at[pl.ds(s * _FPT, _FPT)], out_hbm.at[wid])


@functools.lru_cache(maxsize=None)
def _build_sc_kernels():
    mesh = plsc.VectorSubcoreMesh(core_axis_name="c", subcore_axis_name="s",
                                  num_cores=_NC, num_subcores=_NS)
    aggr = pl.kernel(
        _sc_aggr_body,
        mesh=mesh,
        out_type=jax.ShapeDtypeStruct((_NW, _RPT, _D), jnp.float32),
        scratch_types=[
            pltpu.VMEM((_CPP, _CH), jnp.int32),
            pltpu.VMEM((_CPP, _CH), jnp.int32),
            pltpu.VMEM((_CH, _D), jnp.float32),
            pltpu.VMEM((_CH, _D), jnp.float32),
            pltpu.VMEM_SHARED((_NSLAB, _D), jnp.float32),
            pltpu.SemaphoreType.DMA,
            pltpu.SemaphoreType.DMA,
        ],
    )
    counts = pl.kernel(
        _sc_counts_body,
        mesh=mesh,
        out_type=jax.ShapeDtypeStruct((_NW, _FPT), jnp.float32),
        scratch_types=[
            pltpu.VMEM((_NCH, _CH), jnp.int32),
            pltpu.VMEM((_NCH, _CH), jnp.int32),
            pltpu.VMEM((_NCH, _CH), jnp.int32),
            pltpu.VMEM((_CH,), jnp.int32),
            pltpu.VMEM((_CH,), jnp.int32),
            pltpu.VMEM((_CH,), jnp.float32),
            pltpu.VMEM_SHARED((_NF,), jnp.float32),
        ],
    )
    return aggr, counts


def _sc_aggr(h, src3, dst3, zrows):
    return _build_sc_kernels()[0](h, src3, dst3, zrows)


def _sc_counts(dst3, a03, a13, zflat):
    return _build_sc_kernels()[1](dst3, a03, a13, zflat)


def _tc_embed(x, e1p, e2p):
    bn = 1000

    def body(x_ref, e1_ref, e2_ref, o_ref):
        xb = x_ref[...]
        iota = lax.broadcasted_iota(jnp.int32, (bn, _D), 1)
        oh1 = (iota == xb[:, 0:1]).astype(jnp.float32)
        oh2 = (iota == xb[:, 1:2]).astype(jnp.float32)
        o_ref[...] = (
            jnp.dot(oh1, e1_ref[...], preferred_element_type=jnp.float32,
                    precision=lax.Precision.HIGHEST)
            + jnp.dot(oh2, e2_ref[...], preferred_element_type=jnp.float32,
                      precision=lax.Precision.HIGHEST))

    return pl.pallas_call(
        body,
        grid=(_N // bn,),
        in_specs=[
            pl.BlockSpec((bn, 2), lambda i: (i, 0)),
            pl.BlockSpec((_D, _D), lambda i: (0, 0)),
            pl.BlockSpec((_D, _D), lambda i: (0, 0)),
        ],
        out_specs=pl.BlockSpec((bn, _D), lambda i: (i, 0)),
        out_shape=jax.ShapeDtypeStruct((_N, _D), jnp.float32),
    )(x, e1p, e2p)


_BN = 1000
_NB = _N // _BN


def _tc_layer(aggr2, h, caux2, ecat, w1, b1, w2, b2, g, bt, relu):
    def body(a_ref, h_ref, c_ref, e_ref, w1_ref, b1_ref, w2_ref, b2_ref,
             g_ref, bt_ref, o_ref, sum_sc, sq_sc):
        ph = pl.program_id(0)
        i = pl.program_id(1)

        def compute_z():
            ecat_v = e_ref[...]
            selfvec = ecat_v[4:5, :] + ecat_v[8:9, :]
            a = a_ref[0] + a_ref[1] + h_ref[...] + selfvec
            c16 = c_ref[0] + c_ref[1]
            a = a + jnp.dot(c16, ecat_v, preferred_element_type=jnp.float32,
                            precision=lax.Precision.HIGHEST)
            hid = jnp.dot(a.astype(jnp.bfloat16),
                          w1_ref[...].astype(jnp.bfloat16),
                          preferred_element_type=jnp.float32) + b1_ref[...]
            hid = jnp.maximum(hid, 0.0)
            return jnp.dot(hid.astype(jnp.bfloat16),
                           w2_ref[...].astype(jnp.bfloat16),
                           preferred_element_type=jnp.float32) + b2_ref[...]

        @pl.when(ph == 0)
        def _():
            z = compute_z()
            s1 = jnp.sum(z, 0, keepdims=True)
            s2 = jnp.sum(z * z, 0, keepdims=True)

            @pl.when(i == 0)
            def _():
                sum_sc[...] = s1
                sq_sc[...] = s2

            @pl.when(i > 0)
            def _():
                sum_sc[...] = sum_sc[...] + s1
                sq_sc[...] = sq_sc[...] + s2

        @pl.when(ph == 1)
        def _():
            z = compute_z()
            mean = sum_sc[...] * (1.0 / _N)
            var = sq_sc[...] * (1.0 / _N) - mean * mean
            zn = (z - mean) / jnp.sqrt(var + 1e-5) * g_ref[...] + bt_ref[...]
            if relu:
                zn = jnp.maximum(zn, 0.0)
            o_ref[...] = zn

    return pl.pallas_call(
        body,
        grid=(2, _NB),
        in_specs=[
            pl.BlockSpec((2, _BN, _D), lambda p, i: (0, i, 0)),
            pl.BlockSpec((_BN, _D), lambda p, i: (i, 0)),
            pl.BlockSpec((2, _BN, 16), lambda p, i: (0, i, 0)),
            pl.BlockSpec((16, _D), lambda p, i: (0, 0)),
            pl.BlockSpec((_D, 2 * _D), lambda p, i: (0, 0)),
            pl.BlockSpec((1, 2 * _D), lambda p, i: (0, 0)),
            pl.BlockSpec((2 * _D, _D), lambda p, i: (0, 0)),
            pl.BlockSpec((1, _D), lambda p, i: (0, 0)),
            pl.BlockSpec((1, _D), lambda p, i: (0, 0)),
            pl.BlockSpec((1, _D), lambda p, i: (0, 0)),
        ],
        out_specs=pl.BlockSpec((_BN, _D), lambda p, i: (i, 0)),
        out_shape=jax.ShapeDtypeStruct((_N, _D), jnp.float32),
        scratch_shapes=[
            pltpu.VMEM((1, _D), jnp.float32),
            pltpu.VMEM((1, _D), jnp.float32),
        ],
    )(aggr2, h, caux2, ecat, w1, b1, w2, b2, g, bt)


def kernel(x, edge_index, edge_attr, x_emb1, x_emb2, e_emb1, e_emb2,
           W1, b1, W2, b2, gamma, beta):
    num_layer = W1.shape[0]
    f32 = jnp.float32

    pad = _EPAD - _E
    padi = jnp.arange(pad, dtype=jnp.int32) % 8
    src_p = jnp.concatenate([edge_index[0].astype(jnp.int32), padi])
    dst_p = jnp.concatenate([edge_index[1].astype(jnp.int32), _N + padi])
    a0_p = jnp.concatenate([edge_attr[:, 0].astype(jnp.int32),
                            jnp.zeros((pad,), jnp.int32)])
    a1_p = jnp.concatenate([edge_attr[:, 1].astype(jnp.int32),
                            jnp.zeros((pad,), jnp.int32)])
    src4 = src_p.reshape(_NW, _NPH, _CPP, _CH)
    dst4 = dst_p.reshape(_NW, _NPH, _CPP, _CH)
    dst3 = dst_p.reshape(_NW, _NCH, _CH)
    a03 = a0_p.reshape(_NW, _NCH, _CH)
    a13 = a1_p.reshape(_NW, _NCH, _CH)

    zrows = jnp.zeros((_RPT, _D), f32)
    zflat = jnp.zeros((_FPT,), f32)

    cflat = _sc_counts(dst3, a03, a13, zflat)
    caux2 = cflat.reshape(_NC, _NF // 16, 16)[:, :_N, :]

    e1p = jnp.zeros((_D, _D), f32).at[:x_emb1.shape[0]].set(x_emb1)
    e2p = jnp.zeros((_D, _D), f32).at[:x_emb2.shape[0]].set(x_emb2)
    h = _tc_embed(x.astype(jnp.int32), e1p, e2p)

    zero_rows = jnp.zeros((2, _D), f32)
    for l in range(num_layer):
        slabs = _sc_aggr(h, src4, dst4, zrows)
        aggr2 = slabs.reshape(_NC, _NSLAB, _D)
        ecat = jnp.concatenate(
            [e_emb1[l], zero_rows, e_emb2[l], jnp.zeros((5, _D), f32)], axis=0)
        h = _tc_layer(aggr2, h, caux2, ecat, W1[l], b1[l].reshape(1, -1),
                      W2[l], b2[l].reshape(1, -1), gamma[l].reshape(1, -1),
                      beta[l].reshape(1, -1), relu=(l < num_layer - 1))
    return h
